# Initial kernel scaffold; baseline (speedup 1.0000x reference)
import jax
import jax.numpy as jnp
from jax.experimental import pallas as pl


def kernel(x, labels, centers):
    raise NotImplementedError("write your pallas kernel here")


# speedup vs baseline: 4.5598x; 4.5598x over previous
import jax
import jax.numpy as jnp
from jax.experimental import pallas as pl
from jax.experimental.pallas import tpu as pltpu

_C = 10
_F = 128
_B = 2048


def _body(x_ref, lab_ref, cen_ref, out_ref, s_ref, n_ref):
    i = pl.program_id(0)

    @pl.when(i == 0)
    def _():
        s_ref[...] = jnp.zeros_like(s_ref)
        n_ref[...] = jnp.zeros_like(n_ref)

    x = x_ref[...]
    labels = lab_ref[...]
    onehot = (labels == jax.lax.broadcasted_iota(jnp.int32, (1, _C), 1)
              ).astype(jnp.float32)
    cxy = jax.lax.dot(onehot, cen_ref[...],
                      precision=jax.lax.Precision.HIGHEST,
                      preferred_element_type=jnp.float32)
    diff = x - cxy
    d2 = jnp.sum(diff * diff, axis=1, keepdims=True)
    dist = jnp.sqrt(d2)
    s_ref[...] += jnp.sum(dist * onehot, axis=0, keepdims=True)
    n_ref[...] += jnp.sum(onehot, axis=0, keepdims=True)

    @pl.when(i == pl.num_programs(0) - 1)
    def _():
        s = s_ref[...]
        n = n_ref[...]
        out_ref[...] = jnp.sum(jnp.where(n > 0, s / n, 0.0),
                               axis=1, keepdims=True)


def kernel(x, labels, centers):
    batch = x.shape[0]
    labels2 = labels.astype(jnp.int32).reshape(batch, 1)
    out = pl.pallas_call(
        _body,
        grid=(batch // _B,),
        in_specs=[
            pl.BlockSpec((_B, _F), lambda i: (i, 0)),
            pl.BlockSpec((_B, 1), lambda i: (i, 0)),
            pl.BlockSpec((_C, _F), lambda i: (0, 0)),
        ],
        out_specs=pl.BlockSpec((1, 1), lambda i: (0, 0)),
        out_shape=jax.ShapeDtypeStruct((1, 1), jnp.float32),
        scratch_shapes=[
            pltpu.VMEM((1, _C), jnp.float32),
            pltpu.VMEM((1, _C), jnp.float32),
        ],
        compiler_params=pltpu.CompilerParams(
            dimension_semantics=("arbitrary",)),
    )(x, labels2, centers)
    return out[0, 0]
